# baseline (speedup 1.0000x reference)
import functools
import math

import jax
import jax.numpy as jnp
from jax import lax
from jax.experimental import pallas as pl
from jax.experimental.pallas import tpu as pltpu
from jax.experimental.pallas import tpu_sc as plsc

_B, _N, _D, _H, _K = 32, 8192, 128, 32, 256
_EPS = 1e-5
_L = 16
_NCHUNK = _N // _L
_MIN32 = -2147483648


def _ln(x, g, b):
    m = jnp.mean(x, axis=-1, keepdims=True)
    xc = x - m
    v = jnp.mean(xc * xc, axis=-1, keepdims=True)
    return xc * lax.rsqrt(v + _EPS) * g + b


def _dg(a, b, da, db):
    return lax.dot_general(a, b, (((da,), (db,)), ((), ())),
                           preferred_element_type=jnp.float32)


def _tc_body(q_ref, r_ref,
             qg1_ref, qb1_ref, qW_ref, qb_ref, qg2_ref, qb2_ref,
             rg1_ref, rb1_ref, rW_ref, rb_ref, rg2_ref, rb2_ref,
             scores_ref, cp_ref, c4_ref, scalb_ref, scalg_ref):
    i = pl.program_id(0)

    @pl.when(i == 0)
    def _():
        qn = _ln(q_ref[...], qg1_ref[...], qb1_ref[...])
        qv = jnp.dot(qn, qW_ref[...], preferred_element_type=jnp.float32,
                     precision=lax.Precision.HIGHEST) + qb_ref[...]
        qp = _ln(qv, qg2_ref[...], qb2_ref[...])
        wg = rW_ref[...] * rg1_ref[...]
        gw = jnp.sum(wg, axis=0, keepdims=True)
        bw = _dg(rb1_ref[...], rW_ref[...], 1, 0) + rb_ref[...]
        p = qp * rg2_ref[...] * (1.0 / math.sqrt(_H))
        cp_ref[...] = _dg(p, wg, 1, 1)
        c4_ref[0:1, :] = (_dg(jnp.ones((1, _H), jnp.float32), wg, 1, 1) *
                          (1.0 / _H)).astype(jnp.bfloat16)
        c4_ref[1:2, :] = (_dg(gw, wg, 1, 1) *
                          (2.0 / _H)).astype(jnp.bfloat16)
        c4_ref[2:3, :] = (_dg(bw, wg, 1, 1) *
                          (2.0 / _H)).astype(jnp.bfloat16)
        c4_ref[3:4, :] = jnp.full((1, _D), 1.0 / _D, jnp.bfloat16)
        c4_ref[4:8, :] = jnp.zeros((4, _D), jnp.bfloat16)
        c4_ref[8:40, :] = jnp.transpose(wg).astype(jnp.bfloat16)
        scalb_ref[:, 0:1] = jnp.sum(p * gw, axis=1, keepdims=True)
        scalb_ref[:, 1:2] = jnp.sum(p * bw, axis=1, keepdims=True)
        scalb_ref[:, 2:3] = jnp.sum(p, axis=1, keepdims=True)
        scalb_ref[:, 3:4] = jnp.sum(qp * rb2_ref[...], axis=1,
                                    keepdims=True) * (1.0 / math.sqrt(_H))
        scalb_ref[:, 4:8] = jnp.zeros((_B, 4), jnp.float32)
        scalg_ref[0:1, 0:1] = jnp.sum(gw, keepdims=True) * (1.0 / _H)
        scalg_ref[0:1, 1:2] = jnp.sum(bw, keepdims=True) * (1.0 / _H)
        scalg_ref[0:1, 2:3] = jnp.sum(gw * gw, keepdims=True) * (1.0 / _H)
        scalg_ref[0:1, 3:4] = jnp.sum(gw * bw, keepdims=True) * (1.0 / _H)
        scalg_ref[0:1, 4:5] = jnp.sum(bw * bw, keepdims=True) * (1.0 / _H)
        scalg_ref[0:1, 5:8] = jnp.zeros((1, 3), jnp.float32)

    c4_ref[4:5, :] = cp_ref[pl.ds(i, 1), :].astype(jnp.bfloat16)
    x = r_ref[0]
    xb = x.astype(jnp.bfloat16)
    m5 = _dg(c4_ref[...], xb, 1, 1)
    s2 = _dg(jnp.full((1, _D), 1.0 / _D, jnp.float32), x * x, 1, 1)
    y = m5[8:40, :]
    t2 = _dg(jnp.full((1, _H), 1.0 / _H, jnp.float32), y * y, 1, 0)

    m1 = m5[0:1, :]
    m_g = m5[1:2, :]
    m_b = m5[2:3, :]
    mu = m5[3:4, :]
    m_p = m5[4:5, :]

    sb = scalb_ref[pl.ds(i, 1), :]
    pgw_s, pbw_s, sp_s, qb2_s = (sb[:, 0:1], sb[:, 1:2], sb[:, 2:3],
                                 sb[:, 3:4])
    gl = scalg_ref[...]
    s_g, s_b, g2c, gbc, b2c = (gl[0:1, 0:1], gl[0:1, 1:2], gl[0:1, 2:3],
                               gl[0:1, 3:4], gl[0:1, 4:5])

    a = lax.rsqrt(s2 - mu * mu + _EPS)
    am = a * mu
    nu1 = a * m_p - am * pgw_s + pbw_s
    mu2 = a * m1 - am * s_g + s_b
    syc = m_b - am * m_g
    sc2 = am * am * g2c - 2.0 * am * gbc + b2c
    q2 = a * a * t2 + a * syc + sc2
    inv2 = lax.rsqrt(q2 - mu2 * mu2 + _EPS)
    scores_ref[0] = inv2 * (nu1 - mu2 * sp_s) + qb2_s


def _tc_scores(q, r, q_ln1_g, q_ln1_b, q_W, q_b, q_ln2_g, q_ln2_b,
               r_ln1_g, r_ln1_b, r_W, r_b, r_ln2_g, r_ln2_b):
    row = lambda a: a.reshape(1, -1)
    col = lambda a: a.reshape(-1, 1)
    const = lambda shape: pl.BlockSpec(shape, lambda i: (0,) * len(shape))
    return pl.pallas_call(
        _tc_body,
        grid=(_B,),
        in_specs=[
            const((_B, _D)),
            pl.BlockSpec((1, _N, _D), lambda i: (i, 0, 0)),
            const((1, _D)), const((1, _D)),
            const((_D, _H)), const((1, _H)),
            const((1, _H)), const((1, _H)),
            const((_D, 1)), const((1, _D)),
            const((_D, _H)), const((1, _H)),
            const((1, _H)), const((1, _H)),
        ],
        out_specs=pl.BlockSpec((1, 1, _N), lambda i: (i, 0, 0)),
        out_shape=jax.ShapeDtypeStruct((_B, 1, _N), jnp.float32),
        scratch_shapes=[
            pltpu.VMEM((_B, _D), jnp.float32),
            pltpu.VMEM((48, _D), jnp.bfloat16),
            pltpu.VMEM((_B, 8), jnp.float32),
            pltpu.VMEM((1, 8), jnp.float32),
        ],
    )(q, r, row(q_ln1_g), row(q_ln1_b), q_W, row(q_b), row(q_ln2_g),
      row(q_ln2_b), col(r_ln1_g), row(r_ln1_b), r_W, row(r_b),
      row(r_ln2_g), row(r_ln2_b))


def _sc_body(scores_hbm, rv_hbm, tau_hbm, rs_hbm, sc_hbm, out_hbm,
             s_v, r_v, keys_v, hist_v, tau_v, rs_v, sc_v, res_v, sem):
    wid = lax.axis_index("s") * 2 + lax.axis_index("c")

    pltpu.sync_copy(scores_hbm.at[wid], s_v)
    pltpu.sync_copy(rv_hbm.at[wid], r_v)
    pltpu.sync_copy(tau_hbm, tau_v)
    pltpu.sync_copy(rs_hbm, rs_v)
    pltpu.sync_copy(sc_hbm, sc_v)

    lanes = lax.iota(jnp.int32, _L)
    ones_i = jnp.ones((_L,), jnp.int32)

    def pass_a(c, mx):
        s = s_v[pl.ds(c * _L, _L)]
        b = lax.bitcast_convert_type(s, jnp.int32)
        k = jnp.where(b < 0, b ^ jnp.int32(0x7FFFFFFF), b) ^ jnp.int32(_MIN32)
        keys_v[pl.ds(c * _L, _L)] = k
        return jnp.maximum(mx, s)

    mxv = lax.fori_loop(0, _NCHUNK, pass_a,
                        jnp.full((_L,), -jnp.inf, jnp.float32))
    smax = jnp.max(mxv)

    def level(shift, prefix, c_above, k_rem):
        def zero(j, _):
            hist_v[pl.ds(j * _L, _L)] = jnp.zeros((_L,), jnp.int32)
            return 0

        lax.fori_loop(0, 256, zero, 0)

        def scan(c, _):
            k = keys_v[pl.ds(c * _L, _L)]
            if shift == 24:
                upd = ones_i
            else:
                valid = lax.shift_right_logical(k, shift + 8) == prefix
                upd = jnp.where(valid, ones_i, 0)
            b = jnp.bitwise_and(lax.shift_right_logical(k, shift), 255)
            plsc.addupdate_scatter(hist_v, [b * _L + lanes], upd)
            return 0

        lax.fori_loop(0, _NCHUNK, scan, 0)

        def sscan(j, carry):
            cum, found, chosen, c_ab, c_at = carry
            binno = 255 - j
            cnt = jnp.sum(hist_v[pl.ds(binno * _L, _L)])
            new_cum = cum + cnt
            hit = jnp.logical_and(jnp.logical_not(found), new_cum >= k_rem)
            chosen = jnp.where(hit, binno, chosen)
            c_ab = jnp.where(hit, cum, c_ab)
            c_at = jnp.where(hit, cnt, c_at)
            found = jnp.logical_or(found, hit)
            return new_cum, found, chosen, c_ab, c_at

        z = jnp.int32(0)
        _, _, chosen, c_ab, c_at = lax.fori_loop(
            0, 256, sscan, (z, jnp.bool_(False), z, z, z))

        if shift == 24:
            prefix = chosen
        else:
            prefix = jnp.bitwise_or(prefix * 256, chosen)
        return prefix, c_above + c_ab, k_rem - c_ab, c_at

    prefix, c_above, k_rem = jnp.int32(0), jnp.int32(0), jnp.int32(_K)
    c_at = jnp.int32(0)
    for shift in (24, 16, 8, 0):
        prefix, c_above, k_rem, c_at = level(shift, prefix, c_above, k_rem)

    kt = prefix ^ jnp.int32(_MIN32)
    tb = jnp.where(kt < 0, kt ^ jnp.int32(0x7FFFFFFF), kt)
    t_fv = lax.bitcast_convert_type(jnp.broadcast_to(tb, (_L,)), jnp.float32)

    inv_tau = 1.0 / tau_v[...]
    smax_v = jnp.broadcast_to(smax, (_L,))

    def pass_e(c, carry):
        se, ser, ee, eer = carry
        s = s_v[pl.ds(c * _L, _L)]
        rv = r_v[pl.ds(c * _L, _L)]
        e = jnp.exp((s - smax_v) * inv_tau)
        er = e * rv
        zf = jnp.zeros((_L,), jnp.float32)
        gt = s > t_fv
        eq = s == t_fv
        return (se + jnp.where(gt, e, zf), ser + jnp.where(gt, er, zf),
                ee + jnp.where(eq, e, zf), eer + jnp.where(eq, er, zf))

    zf = jnp.zeros((_L,), jnp.float32)
    se, ser, ee, eer = lax.fori_loop(0, _NCHUNK, pass_e, (zf, zf, zf, zf))
    bv = lambda s: jnp.broadcast_to(s, (_L,))
    frac = (bv(k_rem.astype(jnp.float32)) /
            bv(jnp.maximum(c_at, 1).astype(jnp.float32)))
    den = bv(jnp.sum(se)) + frac * bv(jnp.sum(ee))
    num = bv(jnp.sum(ser)) + frac * bv(jnp.sum(eer))
    pred = num / den

    base = jnp.clip(pred, 0.0001, 1 - 0.0001)
    zq = base / (1.0 - base)
    bi = lax.bitcast_convert_type(zq, jnp.int32)
    ex = lax.shift_right_logical(bi, 23) - 127
    man = lax.bitcast_convert_type(jnp.bitwise_or(jnp.bitwise_and(
        bi, jnp.int32(0x7FFFFF)), jnp.int32(0x3F800000)), jnp.float32)
    big = man > 1.4142135
    man = jnp.where(big, man * 0.5, man)
    ex = ex + jnp.where(big, 1, 0)
    u = (man - 1.0) / (man + 1.0)
    u2 = u * u
    lnm = 2.0 * u * (1.0 + u2 * (1.0 / 3.0 + u2 * (
        1.0 / 5.0 + u2 * (1.0 / 7.0 + u2 * (1.0 / 9.0)))))
    logit = ex.astype(jnp.float32) * 0.6931471805599453 + lnm
    zz = sc_v[...] * logit + rs_v[...]
    res_v[...] = 1.0 / (1.0 + jnp.exp(-zz))
    pltpu.sync_copy(res_v, out_hbm.at[wid])


def _sc_tail(scores, ref_vals, tau, res_scale, scale):
    mesh = plsc.VectorSubcoreMesh(core_axis_name="c", subcore_axis_name="s")
    f = pl.kernel(
        _sc_body,
        out_type=jax.ShapeDtypeStruct((_B, _L), jnp.float32),
        mesh=mesh,
        compiler_params=pltpu.CompilerParams(needs_layout_passes=False),
        scratch_types=[
            pltpu.VMEM((_N,), jnp.float32),
            pltpu.VMEM((_N,), jnp.float32),
            pltpu.VMEM((_N,), jnp.int32),
            pltpu.VMEM((256 * _L,), jnp.int32),
            pltpu.VMEM((_L,), jnp.float32),
            pltpu.VMEM((_L,), jnp.float32),
            pltpu.VMEM((_L,), jnp.float32),
            pltpu.VMEM((_L,), jnp.float32),
            pltpu.SemaphoreType.DMA,
        ],
    )
    tv = jnp.broadcast_to(tau.reshape(1), (_L,))
    rsv = jnp.broadcast_to(res_scale.reshape(1), (_L,))
    scv = jnp.broadcast_to(scale.reshape(1), (_L,))
    return f(scores, ref_vals, tv, rsv, scv)


@jax.jit
def kernel(q, r, ref_vals, tau,
           q_ln1_g, q_ln1_b, q_W, q_b, q_ln2_g, q_ln2_b,
           r_ln1_g, r_ln1_b, r_W, r_b, r_ln2_g, r_ln2_b,
           res_scale, scale):
    scores = _tc_scores(q, r, q_ln1_g, q_ln1_b, q_W, q_b, q_ln2_g, q_ln2_b,
                        r_ln1_g, r_ln1_b, r_W, r_b, r_ln2_g,
                        r_ln2_b).reshape(_B, _N)
    out = _sc_tail(scores, ref_vals, tau, res_scale, scale)
    return out[:, 0]


# speedup vs baseline: 1.0568x; 1.0568x over previous
import functools
import math

import jax
import jax.numpy as jnp
from jax import lax
from jax.experimental import pallas as pl
from jax.experimental.pallas import tpu as pltpu
from jax.experimental.pallas import tpu_sc as plsc

_B, _N, _D, _H, _K = 32, 8192, 128, 32, 256
_EPS = 1e-5
_L = 16
_NCHUNK = _N // _L
_MIN32 = -2147483648


def _ln(x, g, b):
    m = jnp.mean(x, axis=-1, keepdims=True)
    xc = x - m
    v = jnp.mean(xc * xc, axis=-1, keepdims=True)
    return xc * lax.rsqrt(v + _EPS) * g + b


def _dg(a, b, da, db):
    return lax.dot_general(a, b, (((da,), (db,)), ((), ())),
                           preferred_element_type=jnp.float32)


def _tc_body(q_ref, r_ref,
             qg1_ref, qb1_ref, qW_ref, qb_ref, qg2_ref, qb2_ref,
             rg1_ref, rb1_ref, rW_ref, rb_ref, rg2_ref, rb2_ref,
             scores_ref, cp_ref, c4_ref, scalb_ref, scalg_ref):
    i = pl.program_id(0)

    @pl.when(i == 0)
    def _():
        qn = _ln(q_ref[...], qg1_ref[...], qb1_ref[...])
        qv = jnp.dot(qn, qW_ref[...], preferred_element_type=jnp.float32,
                     precision=lax.Precision.HIGHEST) + qb_ref[...]
        qp = _ln(qv, qg2_ref[...], qb2_ref[...])
        wg = rW_ref[...] * rg1_ref[...]
        gw = jnp.sum(wg, axis=0, keepdims=True)
        bw = _dg(rb1_ref[...], rW_ref[...], 1, 0) + rb_ref[...]
        p = qp * rg2_ref[...] * (1.0 / math.sqrt(_H))
        cp_ref[...] = _dg(p, wg, 1, 1)
        c4_ref[0:1, :] = (_dg(jnp.ones((1, _H), jnp.float32), wg, 1, 1) *
                          (1.0 / _H)).astype(jnp.bfloat16)
        c4_ref[1:2, :] = (_dg(gw, wg, 1, 1) *
                          (2.0 / _H)).astype(jnp.bfloat16)
        c4_ref[2:3, :] = (_dg(bw, wg, 1, 1) *
                          (2.0 / _H)).astype(jnp.bfloat16)
        c4_ref[3:4, :] = jnp.full((1, _D), 1.0 / _D, jnp.bfloat16)
        c4_ref[4:8, :] = jnp.zeros((4, _D), jnp.bfloat16)
        c4_ref[8:40, :] = jnp.transpose(wg).astype(jnp.bfloat16)
        scalb_ref[:, 0:1] = jnp.sum(p * gw, axis=1, keepdims=True)
        scalb_ref[:, 1:2] = jnp.sum(p * bw, axis=1, keepdims=True)
        scalb_ref[:, 2:3] = jnp.sum(p, axis=1, keepdims=True)
        scalb_ref[:, 3:4] = jnp.sum(qp * rb2_ref[...], axis=1,
                                    keepdims=True) * (1.0 / math.sqrt(_H))
        scalb_ref[:, 4:8] = jnp.zeros((_B, 4), jnp.float32)
        scalg_ref[0:1, 0:1] = jnp.sum(gw, keepdims=True) * (1.0 / _H)
        scalg_ref[0:1, 1:2] = jnp.sum(bw, keepdims=True) * (1.0 / _H)
        scalg_ref[0:1, 2:3] = jnp.sum(gw * gw, keepdims=True) * (1.0 / _H)
        scalg_ref[0:1, 3:4] = jnp.sum(gw * bw, keepdims=True) * (1.0 / _H)
        scalg_ref[0:1, 4:5] = jnp.sum(bw * bw, keepdims=True) * (1.0 / _H)
        scalg_ref[0:1, 5:8] = jnp.zeros((1, 3), jnp.float32)

    c4_ref[4:5, :] = cp_ref[pl.ds(i, 1), :].astype(jnp.bfloat16)
    x = r_ref[0]
    xb = x.astype(jnp.bfloat16)
    m5 = _dg(c4_ref[...], xb, 1, 1)
    s2 = _dg(jnp.full((1, _D), 1.0 / _D, jnp.float32), x * x, 1, 1)
    y = m5[8:40, :]
    t2 = _dg(jnp.full((1, _H), 1.0 / _H, jnp.float32), y * y, 1, 0)

    m1 = m5[0:1, :]
    m_g = m5[1:2, :]
    m_b = m5[2:3, :]
    mu = m5[3:4, :]
    m_p = m5[4:5, :]

    sb = scalb_ref[pl.ds(i, 1), :]
    pgw_s, pbw_s, sp_s, qb2_s = (sb[:, 0:1], sb[:, 1:2], sb[:, 2:3],
                                 sb[:, 3:4])
    gl = scalg_ref[...]
    s_g, s_b, g2c, gbc, b2c = (gl[0:1, 0:1], gl[0:1, 1:2], gl[0:1, 2:3],
                               gl[0:1, 3:4], gl[0:1, 4:5])

    a = lax.rsqrt(s2 - mu * mu + _EPS)
    am = a * mu
    nu1 = a * m_p - am * pgw_s + pbw_s
    mu2 = a * m1 - am * s_g + s_b
    syc = m_b - am * m_g
    sc2 = am * am * g2c - 2.0 * am * gbc + b2c
    q2 = a * a * t2 + a * syc + sc2
    inv2 = lax.rsqrt(q2 - mu2 * mu2 + _EPS)
    scores_ref[0] = inv2 * (nu1 - mu2 * sp_s) + qb2_s


def _tc_scores(q, r, q_ln1_g, q_ln1_b, q_W, q_b, q_ln2_g, q_ln2_b,
               r_ln1_g, r_ln1_b, r_W, r_b, r_ln2_g, r_ln2_b):
    row = lambda a: a.reshape(1, -1)
    col = lambda a: a.reshape(-1, 1)
    const = lambda shape: pl.BlockSpec(shape, lambda i: (0,) * len(shape))
    return pl.pallas_call(
        _tc_body,
        grid=(_B,),
        in_specs=[
            const((_B, _D)),
            pl.BlockSpec((1, _N, _D), lambda i: (i, 0, 0)),
            const((1, _D)), const((1, _D)),
            const((_D, _H)), const((1, _H)),
            const((1, _H)), const((1, _H)),
            const((_D, 1)), const((1, _D)),
            const((_D, _H)), const((1, _H)),
            const((1, _H)), const((1, _H)),
        ],
        out_specs=pl.BlockSpec((1, 1, _N), lambda i: (i, 0, 0)),
        out_shape=jax.ShapeDtypeStruct((_B, 1, _N), jnp.float32),
        scratch_shapes=[
            pltpu.VMEM((_B, _D), jnp.float32),
            pltpu.VMEM((48, _D), jnp.bfloat16),
            pltpu.VMEM((_B, 8), jnp.float32),
            pltpu.VMEM((1, 8), jnp.float32),
        ],
    )(q, r, row(q_ln1_g), row(q_ln1_b), q_W, row(q_b), row(q_ln2_g),
      row(q_ln2_b), col(r_ln1_g), row(r_ln1_b), r_W, row(r_b),
      row(r_ln2_g), row(r_ln2_b))


def _sc_body(scores_hbm, rv_hbm, tau_hbm, rs_hbm, sc_hbm, out_hbm,
             s_v, r_v, keys_v, hist_v, tau_v, rs_v, sc_v, res_v, sem):
    wid = lax.axis_index("s") * 2 + lax.axis_index("c")

    pltpu.sync_copy(scores_hbm.at[wid], s_v)
    pltpu.sync_copy(rv_hbm.at[wid], r_v)
    pltpu.sync_copy(tau_hbm, tau_v)
    pltpu.sync_copy(rs_hbm, rs_v)
    pltpu.sync_copy(sc_hbm, sc_v)

    lanes = lax.iota(jnp.int32, _L)
    ones_i = jnp.ones((_L,), jnp.int32)
    zero_i = jnp.zeros((_L,), jnp.int32)
    lane_base = lanes * 256

    def zero_hist():
        def zero(j, _):
            hist_v[pl.ds(j * _L, _L)] = zero_i
            return 0
        lax.fori_loop(0, 256, zero, 0, unroll=8)

    zero_hist()

    def pass_a(c, mx):
        s = s_v[pl.ds(c * _L, _L)]
        b = lax.bitcast_convert_type(s, jnp.int32)
        k = jnp.where(b < 0, b ^ jnp.int32(0x7FFFFFFF), b) ^ jnp.int32(_MIN32)
        keys_v[pl.ds(c * _L, _L)] = k
        plsc.addupdate_scatter(
            hist_v, [lane_base + lax.shift_right_logical(k, 24)], ones_i)
        return jnp.maximum(mx, s)

    mxv = lax.fori_loop(0, _NCHUNK, pass_a,
                        jnp.full((_L,), -jnp.inf, jnp.float32), unroll=8)
    smax = jnp.max(mxv)

    bv = lambda s: jnp.broadcast_to(s, (_L,))

    def bin_select(k_rem_v):
        def chunkscan(j, carry):
            cum, found, chosen, c_ab, c_at = carry
            c = 15 - j
            acc = zero_i
            for l in range(_L):
                acc = acc + hist_v[pl.ds(l * 256 + c * _L, _L)]
            racc = lax.rev(acc, (0,))
            rc = plsc.cumsum(racc)
            tot = bv(jnp.sum(acc))
            mask = (cum + rc) >= k_rem_v
            anyhit = bv(jnp.any(mask))
            hit = jnp.logical_and(jnp.logical_not(found),
                                  anyhit)
            ffs = plsc.all_reduce_ffs(mask)
            above = bv(jnp.sum(jnp.where(lanes < ffs, racc, 0)))
            at = bv(jnp.sum(jnp.where(lanes == ffs, racc, 0)))
            chosen = jnp.where(hit, c * _L + 15 - ffs, chosen)
            c_ab = jnp.where(hit, cum + above, c_ab)
            c_at = jnp.where(hit, at, c_at)
            found = jnp.logical_or(found, hit)
            return cum + tot, found, chosen, c_ab, c_at

        fz = jnp.zeros((_L,), jnp.bool_)
        _, _, chosen, c_ab, c_at = lax.fori_loop(
            0, 16, chunkscan, (zero_i, fz, zero_i, zero_i, zero_i))
        return chosen, c_ab, c_at

    k_rem_v = bv(jnp.int32(_K))
    prefix, c_ab, c_at = bin_select(k_rem_v)
    k_rem_v = k_rem_v - c_ab

    for shift in (16, 8, 0):
        zero_hist()

        def scan(c, _, shift=shift, prefix=prefix):
            k = keys_v[pl.ds(c * _L, _L)]
            valid = lax.shift_right_logical(k, shift + 8) == prefix
            b = jnp.bitwise_and(lax.shift_right_logical(k, shift), 255)
            plsc.addupdate_scatter(hist_v, [lane_base + b],
                                   jnp.where(valid, ones_i, zero_i))
            return 0

        lax.fori_loop(0, _NCHUNK, scan, 0, unroll=8)
        chosen, c_ab, c_at = bin_select(k_rem_v)
        prefix = prefix * 256 + chosen
        k_rem_v = k_rem_v - c_ab

    kt = prefix ^ jnp.int32(_MIN32)
    tb = jnp.where(kt < 0, kt ^ jnp.int32(0x7FFFFFFF), kt)
    t_fv = lax.bitcast_convert_type(tb, jnp.float32)

    inv_tau = 1.0 / tau_v[...]
    smax_v = jnp.broadcast_to(smax, (_L,))

    def pass_e(c, carry):
        se, ser, ee, eer = carry
        s = s_v[pl.ds(c * _L, _L)]
        rv = r_v[pl.ds(c * _L, _L)]
        e = jnp.exp((s - smax_v) * inv_tau)
        er = e * rv
        zf = jnp.zeros((_L,), jnp.float32)
        gt = s > t_fv
        eq = s == t_fv
        return (se + jnp.where(gt, e, zf), ser + jnp.where(gt, er, zf),
                ee + jnp.where(eq, e, zf), eer + jnp.where(eq, er, zf))

    zf = jnp.zeros((_L,), jnp.float32)
    se, ser, ee, eer = lax.fori_loop(0, _NCHUNK, pass_e, (zf, zf, zf, zf),
                                     unroll=8)
    frac = (k_rem_v.astype(jnp.float32) /
            jnp.maximum(c_at, 1).astype(jnp.float32))
    den = bv(jnp.sum(se)) + frac * bv(jnp.sum(ee))
    num = bv(jnp.sum(ser)) + frac * bv(jnp.sum(eer))
    pred = num / den

    base = jnp.clip(pred, 0.0001, 1 - 0.0001)
    zq = base / (1.0 - base)
    bi = lax.bitcast_convert_type(zq, jnp.int32)
    ex = lax.shift_right_logical(bi, 23) - 127
    man = lax.bitcast_convert_type(jnp.bitwise_or(jnp.bitwise_and(
        bi, jnp.int32(0x7FFFFF)), jnp.int32(0x3F800000)), jnp.float32)
    big = man > 1.4142135
    man = jnp.where(big, man * 0.5, man)
    ex = ex + jnp.where(big, 1, 0)
    u = (man - 1.0) / (man + 1.0)
    u2 = u * u
    lnm = 2.0 * u * (1.0 + u2 * (1.0 / 3.0 + u2 * (
        1.0 / 5.0 + u2 * (1.0 / 7.0 + u2 * (1.0 / 9.0)))))
    logit = ex.astype(jnp.float32) * 0.6931471805599453 + lnm
    zz = sc_v[...] * logit + rs_v[...]
    res_v[...] = 1.0 / (1.0 + jnp.exp(-zz))
    pltpu.sync_copy(res_v, out_hbm.at[wid])


def _sc_tail(scores, ref_vals, tau, res_scale, scale):
    mesh = plsc.VectorSubcoreMesh(core_axis_name="c", subcore_axis_name="s")
    f = pl.kernel(
        _sc_body,
        out_type=jax.ShapeDtypeStruct((_B, _L), jnp.float32),
        mesh=mesh,
        compiler_params=pltpu.CompilerParams(needs_layout_passes=False),
        scratch_types=[
            pltpu.VMEM((_N,), jnp.float32),
            pltpu.VMEM((_N,), jnp.float32),
            pltpu.VMEM((_N,), jnp.int32),
            pltpu.VMEM((256 * _L,), jnp.int32),
            pltpu.VMEM((_L,), jnp.float32),
            pltpu.VMEM((_L,), jnp.float32),
            pltpu.VMEM((_L,), jnp.float32),
            pltpu.VMEM((_L,), jnp.float32),
            pltpu.SemaphoreType.DMA,
        ],
    )
    tv = jnp.broadcast_to(tau.reshape(1), (_L,))
    rsv = jnp.broadcast_to(res_scale.reshape(1), (_L,))
    scv = jnp.broadcast_to(scale.reshape(1), (_L,))
    return f(scores, ref_vals, tv, rsv, scv)


@jax.jit
def kernel(q, r, ref_vals, tau,
           q_ln1_g, q_ln1_b, q_W, q_b, q_ln2_g, q_ln2_b,
           r_ln1_g, r_ln1_b, r_W, r_b, r_ln2_g, r_ln2_b,
           res_scale, scale):
    scores = _tc_scores(q, r, q_ln1_g, q_ln1_b, q_W, q_b, q_ln2_g, q_ln2_b,
                        r_ln1_g, r_ln1_b, r_W, r_b, r_ln2_g,
                        r_ln2_b).reshape(_B, _N)
    out = _sc_tail(scores, ref_vals, tau, res_scale, scale)
    return out[:, 0]


# speedup vs baseline: 1.0810x; 1.0229x over previous
import functools
import math

import jax
import jax.numpy as jnp
from jax import lax
from jax.experimental import pallas as pl
from jax.experimental.pallas import tpu as pltpu
from jax.experimental.pallas import tpu_sc as plsc

_B, _N, _D, _H, _K = 32, 8192, 128, 32, 256
_EPS = 1e-5
_L = 16
_NCHUNK = _N // _L
_MIN32 = -2147483648


def _ln(x, g, b):
    m = jnp.mean(x, axis=-1, keepdims=True)
    xc = x - m
    v = jnp.mean(xc * xc, axis=-1, keepdims=True)
    return xc * lax.rsqrt(v + _EPS) * g + b


def _dg(a, b, da, db):
    return lax.dot_general(a, b, (((da,), (db,)), ((), ())),
                           preferred_element_type=jnp.float32)


def _tc_body(q_ref, r_ref,
             qg1_ref, qb1_ref, qW_ref, qb_ref, qg2_ref, qb2_ref,
             rg1_ref, rb1_ref, rW_ref, rb_ref, rg2_ref, rb2_ref,
             scores_ref, cp_ref, c4_ref, scalb_ref, scalg_ref):
    i = pl.program_id(0)

    @pl.when(i == 0)
    def _():
        qn = _ln(q_ref[...], qg1_ref[...], qb1_ref[...])
        qv = jnp.dot(qn, qW_ref[...], preferred_element_type=jnp.float32,
                     precision=lax.Precision.HIGHEST) + qb_ref[...]
        qp = _ln(qv, qg2_ref[...], qb2_ref[...])
        wg = rW_ref[...] * rg1_ref[...]
        gw = jnp.sum(wg, axis=0, keepdims=True)
        bw = _dg(rb1_ref[...], rW_ref[...], 1, 0) + rb_ref[...]
        p = qp * rg2_ref[...] * (1.0 / math.sqrt(_H))
        cp_ref[...] = _dg(p, wg, 1, 1)
        c4_ref[0:1, :] = (_dg(jnp.ones((1, _H), jnp.float32), wg, 1, 1) *
                          (1.0 / _H)).astype(jnp.bfloat16)
        c4_ref[1:2, :] = (_dg(gw, wg, 1, 1) *
                          (2.0 / _H)).astype(jnp.bfloat16)
        c4_ref[2:3, :] = (_dg(bw, wg, 1, 1) *
                          (2.0 / _H)).astype(jnp.bfloat16)
        c4_ref[3:4, :] = jnp.full((1, _D), 1.0 / _D, jnp.bfloat16)
        c4_ref[4:8, :] = jnp.zeros((4, _D), jnp.bfloat16)
        c4_ref[8:40, :] = jnp.transpose(wg).astype(jnp.bfloat16)
        scalb_ref[:, 0:1] = jnp.sum(p * gw, axis=1, keepdims=True)
        scalb_ref[:, 1:2] = jnp.sum(p * bw, axis=1, keepdims=True)
        scalb_ref[:, 2:3] = jnp.sum(p, axis=1, keepdims=True)
        scalb_ref[:, 3:4] = jnp.sum(qp * rb2_ref[...], axis=1,
                                    keepdims=True) * (1.0 / math.sqrt(_H))
        scalb_ref[:, 4:8] = jnp.zeros((_B, 4), jnp.float32)
        scalg_ref[0:1, 0:1] = jnp.sum(gw, keepdims=True) * (1.0 / _H)
        scalg_ref[0:1, 1:2] = jnp.sum(bw, keepdims=True) * (1.0 / _H)
        scalg_ref[0:1, 2:3] = jnp.sum(gw * gw, keepdims=True) * (1.0 / _H)
        scalg_ref[0:1, 3:4] = jnp.sum(gw * bw, keepdims=True) * (1.0 / _H)
        scalg_ref[0:1, 4:5] = jnp.sum(bw * bw, keepdims=True) * (1.0 / _H)
        scalg_ref[0:1, 5:8] = jnp.zeros((1, 3), jnp.float32)

    c4_ref[4:5, :] = cp_ref[pl.ds(i, 1), :].astype(jnp.bfloat16)
    x = r_ref[0]
    xb = x.astype(jnp.bfloat16)
    m5 = _dg(c4_ref[...], xb, 1, 1)
    s2 = _dg(jnp.full((1, _D), 1.0 / _D, jnp.float32), x * x, 1, 1)
    y = m5[8:40, :]
    t2 = _dg(jnp.full((1, _H), 1.0 / _H, jnp.float32), y * y, 1, 0)

    m1 = m5[0:1, :]
    m_g = m5[1:2, :]
    m_b = m5[2:3, :]
    mu = m5[3:4, :]
    m_p = m5[4:5, :]

    sb = scalb_ref[pl.ds(i, 1), :]
    pgw_s, pbw_s, sp_s, qb2_s = (sb[:, 0:1], sb[:, 1:2], sb[:, 2:3],
                                 sb[:, 3:4])
    gl = scalg_ref[...]
    s_g, s_b, g2c, gbc, b2c = (gl[0:1, 0:1], gl[0:1, 1:2], gl[0:1, 2:3],
                               gl[0:1, 3:4], gl[0:1, 4:5])

    a = lax.rsqrt(s2 - mu * mu + _EPS)
    am = a * mu
    nu1 = a * m_p - am * pgw_s + pbw_s
    mu2 = a * m1 - am * s_g + s_b
    syc = m_b - am * m_g
    sc2 = am * am * g2c - 2.0 * am * gbc + b2c
    q2 = a * a * t2 + a * syc + sc2
    inv2 = lax.rsqrt(q2 - mu2 * mu2 + _EPS)
    scores_ref[0] = inv2 * (nu1 - mu2 * sp_s) + qb2_s


def _tc_scores(q, r, q_ln1_g, q_ln1_b, q_W, q_b, q_ln2_g, q_ln2_b,
               r_ln1_g, r_ln1_b, r_W, r_b, r_ln2_g, r_ln2_b):
    row = lambda a: a.reshape(1, -1)
    col = lambda a: a.reshape(-1, 1)
    const = lambda shape: pl.BlockSpec(shape, lambda i: (0,) * len(shape))
    return pl.pallas_call(
        _tc_body,
        grid=(_B,),
        in_specs=[
            const((_B, _D)),
            pl.BlockSpec((1, _N, _D), lambda i: (i, 0, 0)),
            const((1, _D)), const((1, _D)),
            const((_D, _H)), const((1, _H)),
            const((1, _H)), const((1, _H)),
            const((_D, 1)), const((1, _D)),
            const((_D, _H)), const((1, _H)),
            const((1, _H)), const((1, _H)),
        ],
        out_specs=pl.BlockSpec((1, 1, _N), lambda i: (i, 0, 0)),
        out_shape=jax.ShapeDtypeStruct((_B, 1, _N), jnp.float32),
        scratch_shapes=[
            pltpu.VMEM((_B, _D), jnp.float32),
            pltpu.VMEM((48, _D), jnp.bfloat16),
            pltpu.VMEM((_B, 8), jnp.float32),
            pltpu.VMEM((1, 8), jnp.float32),
        ],
    )(q, r, row(q_ln1_g), row(q_ln1_b), q_W, row(q_b), row(q_ln2_g),
      row(q_ln2_b), col(r_ln1_g), row(r_ln1_b), r_W, row(r_b),
      row(r_ln2_g), row(r_ln2_b))


def _sc_body(scores_hbm, rv_hbm, tau_hbm, rs_hbm, sc_hbm, out_hbm,
             s_v, r_v, keys_v, ck_v, ck2_v, hist_v, tau_v, rs_v, sc_v,
             res_v, sem):
    wid = lax.axis_index("s") * 2 + lax.axis_index("c")

    pltpu.sync_copy(scores_hbm.at[wid], s_v)
    pltpu.sync_copy(rv_hbm.at[wid], r_v)
    pltpu.sync_copy(tau_hbm, tau_v)
    pltpu.sync_copy(rs_hbm, rs_v)
    pltpu.sync_copy(sc_hbm, sc_v)

    lanes = lax.iota(jnp.int32, _L)
    ones_i = jnp.ones((_L,), jnp.int32)
    zero_i = jnp.zeros((_L,), jnp.int32)
    lane_base = lanes * 256

    def zero_hist():
        def zero(j, _):
            hist_v[pl.ds(j * _L, _L)] = zero_i
            return 0
        lax.fori_loop(0, 256, zero, 0, unroll=8)

    zero_hist()

    def pass_a(c, mx):
        s = s_v[pl.ds(c * _L, _L)]
        b = lax.bitcast_convert_type(s, jnp.int32)
        k = jnp.where(b < 0, b ^ jnp.int32(0x7FFFFFFF), b) ^ jnp.int32(_MIN32)
        keys_v[pl.ds(c * _L, _L)] = k
        plsc.addupdate_scatter(
            hist_v, [lane_base + lax.shift_right_logical(k, 24)], ones_i)
        return jnp.maximum(mx, s)

    mxv = lax.fori_loop(0, _NCHUNK, pass_a,
                        jnp.full((_L,), -jnp.inf, jnp.float32), unroll=8)
    smax = jnp.max(mxv)

    bv = lambda s: jnp.broadcast_to(s, (_L,))

    def bin_select(k_rem_v):
        def chunkscan(j, carry):
            cum, found, chosen, c_ab, c_at = carry
            c = 15 - j
            acc = zero_i
            for l in range(_L):
                acc = acc + hist_v[pl.ds(l * 256 + c * _L, _L)]
            racc = lax.rev(acc, (0,))
            rc = plsc.cumsum(racc)
            tot = bv(jnp.sum(acc))
            mask = (cum + rc) >= k_rem_v
            anyhit = bv(jnp.any(mask))
            hit = jnp.logical_and(jnp.logical_not(found),
                                  anyhit)
            ffs = plsc.all_reduce_ffs(mask)
            above = bv(jnp.sum(jnp.where(lanes < ffs, racc, 0)))
            at = bv(jnp.sum(jnp.where(lanes == ffs, racc, 0)))
            chosen = jnp.where(hit, c * _L + 15 - ffs, chosen)
            c_ab = jnp.where(hit, cum + above, c_ab)
            c_at = jnp.where(hit, at, c_at)
            found = jnp.logical_or(found, hit)
            return cum + tot, found, chosen, c_ab, c_at

        fz = jnp.zeros((_L,), jnp.bool_)
        _, _, chosen, c_ab, c_at = lax.fori_loop(
            0, 16, chunkscan, (zero_i, fz, zero_i, zero_i, zero_i))
        return chosen, c_ab, c_at

    k_rem_v = bv(jnp.int32(_K))
    prefix, c_ab, c_at = bin_select(k_rem_v)
    k_rem_v = k_rem_v - c_ab

    zero_hist()

    def compact1(c, cnt):
        k = keys_v[pl.ds(c * _L, _L)]
        m = lax.shift_right_logical(k, 24) == prefix
        mi = m.astype(jnp.int32)
        pos = cnt + plsc.cumsum(mi) - 1
        plsc.store_scatter(ck_v, [pos], k, mask=m)
        plsc.addupdate_scatter(
            hist_v,
            [lane_base + jnp.bitwise_and(lax.shift_right_logical(k, 16),
                                         255)],
            mi)
        return cnt + jnp.sum(mi)

    cnt1 = lax.fori_loop(0, _NCHUNK, compact1, jnp.int32(0), unroll=8)
    chosen, c_ab, c_at = bin_select(k_rem_v)
    prefix = prefix * 256 + chosen
    k_rem_v = k_rem_v - c_ab

    zero_hist()
    t1 = lax.shift_right_logical(cnt1 + (_L - 1), 4)

    def compact2(c, cnt):
        k = ck_v[pl.ds(c * _L, _L)]
        inb = (c * _L + lanes) < cnt1
        m = jnp.logical_and(inb, lax.shift_right_logical(k, 16) == prefix)
        mi = m.astype(jnp.int32)
        pos = cnt + plsc.cumsum(mi) - 1
        plsc.store_scatter(ck2_v, [pos], k, mask=m)
        plsc.addupdate_scatter(
            hist_v,
            [lane_base + jnp.bitwise_and(lax.shift_right_logical(k, 8),
                                         255)],
            mi)
        return cnt + jnp.sum(mi)

    cnt2 = lax.fori_loop(0, t1, compact2, jnp.int32(0))
    chosen, c_ab, c_at = bin_select(k_rem_v)
    prefix = prefix * 256 + chosen
    k_rem_v = k_rem_v - c_ab

    zero_hist()
    t2 = lax.shift_right_logical(cnt2 + (_L - 1), 4)

    def scan4(c, _):
        k = ck2_v[pl.ds(c * _L, _L)]
        inb = (c * _L + lanes) < cnt2
        m = jnp.logical_and(inb, lax.shift_right_logical(k, 8) == prefix)
        plsc.addupdate_scatter(hist_v, [lane_base + jnp.bitwise_and(k, 255)],
                               m.astype(jnp.int32))
        return 0

    lax.fori_loop(0, t2, scan4, 0)
    chosen, c_ab, c_at = bin_select(k_rem_v)
    prefix = prefix * 256 + chosen
    k_rem_v = k_rem_v - c_ab

    kt = prefix ^ jnp.int32(_MIN32)
    tb = jnp.where(kt < 0, kt ^ jnp.int32(0x7FFFFFFF), kt)
    t_fv = lax.bitcast_convert_type(tb, jnp.float32)

    inv_tau = 1.0 / tau_v[...]
    smax_v = jnp.broadcast_to(smax, (_L,))

    def pass_e(c, carry):
        se, ser, ee, eer = carry
        s = s_v[pl.ds(c * _L, _L)]
        rv = r_v[pl.ds(c * _L, _L)]
        e = jnp.exp((s - smax_v) * inv_tau)
        er = e * rv
        zf = jnp.zeros((_L,), jnp.float32)
        gt = s > t_fv
        eq = s == t_fv
        return (se + jnp.where(gt, e, zf), ser + jnp.where(gt, er, zf),
                ee + jnp.where(eq, e, zf), eer + jnp.where(eq, er, zf))

    zf = jnp.zeros((_L,), jnp.float32)
    se, ser, ee, eer = lax.fori_loop(0, _NCHUNK, pass_e, (zf, zf, zf, zf),
                                     unroll=8)
    frac = (k_rem_v.astype(jnp.float32) /
            jnp.maximum(c_at, 1).astype(jnp.float32))
    den = bv(jnp.sum(se)) + frac * bv(jnp.sum(ee))
    num = bv(jnp.sum(ser)) + frac * bv(jnp.sum(eer))
    pred = num / den

    base = jnp.clip(pred, 0.0001, 1 - 0.0001)
    zq = base / (1.0 - base)
    bi = lax.bitcast_convert_type(zq, jnp.int32)
    ex = lax.shift_right_logical(bi, 23) - 127
    man = lax.bitcast_convert_type(jnp.bitwise_or(jnp.bitwise_and(
        bi, jnp.int32(0x7FFFFF)), jnp.int32(0x3F800000)), jnp.float32)
    big = man > 1.4142135
    man = jnp.where(big, man * 0.5, man)
    ex = ex + jnp.where(big, 1, 0)
    u = (man - 1.0) / (man + 1.0)
    u2 = u * u
    lnm = 2.0 * u * (1.0 + u2 * (1.0 / 3.0 + u2 * (
        1.0 / 5.0 + u2 * (1.0 / 7.0 + u2 * (1.0 / 9.0)))))
    logit = ex.astype(jnp.float32) * 0.6931471805599453 + lnm
    zz = sc_v[...] * logit + rs_v[...]
    res_v[...] = 1.0 / (1.0 + jnp.exp(-zz))
    pltpu.sync_copy(res_v, out_hbm.at[wid])


def _sc_tail(scores, ref_vals, tau, res_scale, scale):
    mesh = plsc.VectorSubcoreMesh(core_axis_name="c", subcore_axis_name="s")
    f = pl.kernel(
        _sc_body,
        out_type=jax.ShapeDtypeStruct((_B, _L), jnp.float32),
        mesh=mesh,
        compiler_params=pltpu.CompilerParams(needs_layout_passes=False),
        scratch_types=[
            pltpu.VMEM((_N,), jnp.float32),
            pltpu.VMEM((_N,), jnp.float32),
            pltpu.VMEM((_N,), jnp.int32),
            pltpu.VMEM((_N,), jnp.int32),
            pltpu.VMEM((_N,), jnp.int32),
            pltpu.VMEM((256 * _L,), jnp.int32),
            pltpu.VMEM((_L,), jnp.float32),
            pltpu.VMEM((_L,), jnp.float32),
            pltpu.VMEM((_L,), jnp.float32),
            pltpu.VMEM((_L,), jnp.float32),
            pltpu.SemaphoreType.DMA,
        ],
    )
    tv = jnp.broadcast_to(tau.reshape(1), (_L,))
    rsv = jnp.broadcast_to(res_scale.reshape(1), (_L,))
    scv = jnp.broadcast_to(scale.reshape(1), (_L,))
    return f(scores, ref_vals, tv, rsv, scv)


@jax.jit
def kernel(q, r, ref_vals, tau,
           q_ln1_g, q_ln1_b, q_W, q_b, q_ln2_g, q_ln2_b,
           r_ln1_g, r_ln1_b, r_W, r_b, r_ln2_g, r_ln2_b,
           res_scale, scale):
    scores = _tc_scores(q, r, q_ln1_g, q_ln1_b, q_W, q_b, q_ln2_g, q_ln2_b,
                        r_ln1_g, r_ln1_b, r_W, r_b, r_ln2_g,
                        r_ln2_b).reshape(_B, _N)
    out = _sc_tail(scores, ref_vals, tau, res_scale, scale)
    return out[:, 0]


# speedup vs baseline: 1.0812x; 1.0001x over previous
import math

import jax
import jax.numpy as jnp
from jax import lax
from jax.experimental import pallas as pl
from jax.experimental.pallas import tpu as pltpu
from jax.experimental.pallas import tpu_sc as plsc

_B, _N, _D, _H, _K = 32, 8192, 128, 32, 256
_EPS = 1e-5
_L = 16
_NCHUNK = _N // _L
_MIN32 = -2147483648


def _ln(x, g, b):
    m = jnp.mean(x, axis=-1, keepdims=True)
    xc = x - m
    v = jnp.mean(xc * xc, axis=-1, keepdims=True)
    return xc * lax.rsqrt(v + _EPS) * g + b


def _dg(a, b, da, db):
    return lax.dot_general(a, b, (((da,), (db,)), ((), ())),
                           preferred_element_type=jnp.float32)


def _tc_body(q_ref, r_ref,
             qg1_ref, qb1_ref, qW_ref, qb_ref, qg2_ref, qb2_ref,
             rg1_ref, rb1_ref, rW_ref, rb_ref, rg2_ref, rb2_ref,
             scores_ref, cp_ref, c4_ref, scalb_ref, scalg_ref):
    i = pl.program_id(0)

    @pl.when(i == 0)
    def _():
        qn = _ln(q_ref[...], qg1_ref[...], qb1_ref[...])
        qv = jnp.dot(qn, qW_ref[...], preferred_element_type=jnp.float32,
                     precision=lax.Precision.HIGHEST) + qb_ref[...]
        qp = _ln(qv, qg2_ref[...], qb2_ref[...])
        wg = rW_ref[...] * rg1_ref[...]
        gw = jnp.sum(wg, axis=0, keepdims=True)
        bw = _dg(rb1_ref[...], rW_ref[...], 1, 0) + rb_ref[...]
        p = qp * rg2_ref[...] * (1.0 / math.sqrt(_H))
        cp_ref[...] = _dg(p, wg, 1, 1)
        c4_ref[0:1, :] = (_dg(jnp.ones((1, _H), jnp.float32), wg, 1, 1) *
                          (1.0 / _H)).astype(jnp.bfloat16)
        c4_ref[1:2, :] = (_dg(gw, wg, 1, 1) *
                          (2.0 / _H)).astype(jnp.bfloat16)
        c4_ref[2:3, :] = (_dg(bw, wg, 1, 1) *
                          (2.0 / _H)).astype(jnp.bfloat16)
        c4_ref[3:4, :] = jnp.full((1, _D), 1.0 / _D, jnp.bfloat16)
        c4_ref[4:8, :] = jnp.zeros((4, _D), jnp.bfloat16)
        c4_ref[8:40, :] = jnp.transpose(wg).astype(jnp.bfloat16)
        scalb_ref[:, 0:1] = jnp.sum(p * gw, axis=1, keepdims=True)
        scalb_ref[:, 1:2] = jnp.sum(p * bw, axis=1, keepdims=True)
        scalb_ref[:, 2:3] = jnp.sum(p, axis=1, keepdims=True)
        scalb_ref[:, 3:4] = jnp.sum(qp * rb2_ref[...], axis=1,
                                    keepdims=True) * (1.0 / math.sqrt(_H))
        scalb_ref[:, 4:8] = jnp.zeros((_B, 4), jnp.float32)
        scalg_ref[0:1, 0:1] = jnp.sum(gw, keepdims=True) * (1.0 / _H)
        scalg_ref[0:1, 1:2] = jnp.sum(bw, keepdims=True) * (1.0 / _H)
        scalg_ref[0:1, 2:3] = jnp.sum(gw * gw, keepdims=True) * (1.0 / _H)
        scalg_ref[0:1, 3:4] = jnp.sum(gw * bw, keepdims=True) * (1.0 / _H)
        scalg_ref[0:1, 4:5] = jnp.sum(bw * bw, keepdims=True) * (1.0 / _H)
        scalg_ref[0:1, 5:8] = jnp.zeros((1, 3), jnp.float32)

    c4_ref[4:5, :] = cp_ref[pl.ds(i, 1), :].astype(jnp.bfloat16)
    x = r_ref[0]
    xb = x.astype(jnp.bfloat16)
    m5 = _dg(c4_ref[...], xb, 1, 1)
    s2 = _dg(jnp.full((1, _D), 1.0 / _D, jnp.float32), x * x, 1, 1)
    y = m5[8:40, :]
    t2 = _dg(jnp.full((1, _H), 1.0 / _H, jnp.float32), y * y, 1, 0)

    m1 = m5[0:1, :]
    m_g = m5[1:2, :]
    m_b = m5[2:3, :]
    mu = m5[3:4, :]
    m_p = m5[4:5, :]

    sb = scalb_ref[pl.ds(i, 1), :]
    pgw_s, pbw_s, sp_s, qb2_s = (sb[:, 0:1], sb[:, 1:2], sb[:, 2:3],
                                 sb[:, 3:4])
    gl = scalg_ref[...]
    s_g, s_b, g2c, gbc, b2c = (gl[0:1, 0:1], gl[0:1, 1:2], gl[0:1, 2:3],
                               gl[0:1, 3:4], gl[0:1, 4:5])

    a = lax.rsqrt(s2 - mu * mu + _EPS)
    am = a * mu
    nu1 = a * m_p - am * pgw_s + pbw_s
    mu2 = a * m1 - am * s_g + s_b
    syc = m_b - am * m_g
    sc2 = am * am * g2c - 2.0 * am * gbc + b2c
    q2 = a * a * t2 + a * syc + sc2
    inv2 = lax.rsqrt(q2 - mu2 * mu2 + _EPS)
    scores_ref[0] = inv2 * (nu1 - mu2 * sp_s) + qb2_s


def _tc_scores(q, r, q_ln1_g, q_ln1_b, q_W, q_b, q_ln2_g, q_ln2_b,
               r_ln1_g, r_ln1_b, r_W, r_b, r_ln2_g, r_ln2_b):
    row = lambda a: a.reshape(1, -1)
    col = lambda a: a.reshape(-1, 1)
    const = lambda shape: pl.BlockSpec(shape, lambda i: (0,) * len(shape))
    return pl.pallas_call(
        _tc_body,
        grid=(_B,),
        in_specs=[
            const((_B, _D)),
            pl.BlockSpec((1, _N, _D), lambda i: (i, 0, 0)),
            const((1, _D)), const((1, _D)),
            const((_D, _H)), const((1, _H)),
            const((1, _H)), const((1, _H)),
            const((_D, 1)), const((1, _D)),
            const((_D, _H)), const((1, _H)),
            const((1, _H)), const((1, _H)),
        ],
        out_specs=pl.BlockSpec((1, 1, _N), lambda i: (i, 0, 0)),
        out_shape=jax.ShapeDtypeStruct((_B, 1, _N), jnp.float32),
        scratch_shapes=[
            pltpu.VMEM((_B, _D), jnp.float32),
            pltpu.VMEM((48, _D), jnp.bfloat16),
            pltpu.VMEM((_B, 8), jnp.float32),
            pltpu.VMEM((1, 8), jnp.float32),
        ],
    )(q, r, row(q_ln1_g), row(q_ln1_b), q_W, row(q_b), row(q_ln2_g),
      row(q_ln2_b), col(r_ln1_g), row(r_ln1_b), r_W, row(r_b),
      row(r_ln2_g), row(r_ln2_b))


def _sc_body(scores_hbm, rv_hbm, tau_hbm, rs_hbm, sc_hbm, out_hbm,
             s_v, r_v, keys_v, ck_v, ck2_v, hist_v, tau_v, rs_v, sc_v,
             res_v, sem):
    wid = lax.axis_index("s") * 2 + lax.axis_index("c")

    pltpu.sync_copy(scores_hbm.at[wid], s_v)
    pltpu.sync_copy(rv_hbm.at[wid], r_v)
    pltpu.sync_copy(tau_hbm, tau_v)
    pltpu.sync_copy(rs_hbm, rs_v)
    pltpu.sync_copy(sc_hbm, sc_v)

    lanes = lax.iota(jnp.int32, _L)
    ones_i = jnp.ones((_L,), jnp.int32)
    zero_i = jnp.zeros((_L,), jnp.int32)
    lane_base = lanes * 256

    def zero_hist():
        def zero(j, _):
            hist_v[pl.ds(j * _L, _L)] = zero_i
            return 0
        lax.fori_loop(0, 256, zero, 0, unroll=8)

    zero_hist()

    def pass_a(c, mx):
        s = s_v[pl.ds(c * _L, _L)]
        b = lax.bitcast_convert_type(s, jnp.int32)
        k = jnp.where(b < 0, b ^ jnp.int32(0x7FFFFFFF), b) ^ jnp.int32(_MIN32)
        keys_v[pl.ds(c * _L, _L)] = k
        plsc.addupdate_scatter(
            hist_v, [lane_base + lax.shift_right_logical(k, 24)], ones_i)
        return jnp.maximum(mx, s)

    mxv = lax.fori_loop(0, _NCHUNK, pass_a,
                        jnp.full((_L,), -jnp.inf, jnp.float32), unroll=8)
    smax = jnp.max(mxv)

    bv = lambda s: jnp.broadcast_to(s, (_L,))

    def bin_select(k_rem_v):
        def chunkscan(j, carry):
            cum, found, chosen, c_ab, c_at = carry
            c = 15 - j
            acc = zero_i
            for l in range(_L):
                acc = acc + hist_v[pl.ds(l * 256 + c * _L, _L)]
            racc = lax.rev(acc, (0,))
            rc = plsc.cumsum(racc)
            tot = bv(jnp.sum(acc))
            mask = (cum + rc) >= k_rem_v
            anyhit = bv(jnp.any(mask))
            hit = jnp.logical_and(jnp.logical_not(found),
                                  anyhit)
            ffs = plsc.all_reduce_ffs(mask)
            above = bv(jnp.sum(jnp.where(lanes < ffs, racc, 0)))
            at = bv(jnp.sum(jnp.where(lanes == ffs, racc, 0)))
            chosen = jnp.where(hit, c * _L + 15 - ffs, chosen)
            c_ab = jnp.where(hit, cum + above, c_ab)
            c_at = jnp.where(hit, at, c_at)
            found = jnp.logical_or(found, hit)
            return cum + tot, found, chosen, c_ab, c_at

        fz = jnp.zeros((_L,), jnp.bool_)
        _, _, chosen, c_ab, c_at = lax.fori_loop(
            0, 16, chunkscan, (zero_i, fz, zero_i, zero_i, zero_i))
        return chosen, c_ab, c_at

    k_rem_v = bv(jnp.int32(_K))
    prefix, c_ab, c_at = bin_select(k_rem_v)
    k_rem_v = k_rem_v - c_ab

    zero_hist()

    def compact1(c, cnt):
        k = keys_v[pl.ds(c * _L, _L)]
        m = lax.shift_right_logical(k, 24) == prefix
        mi = m.astype(jnp.int32)
        pos = cnt + plsc.cumsum(mi) - 1
        plsc.store_scatter(ck_v, [pos], k, mask=m)
        plsc.addupdate_scatter(
            hist_v,
            [lane_base + jnp.bitwise_and(lax.shift_right_logical(k, 16),
                                         255)],
            mi)
        return cnt + jnp.sum(mi)

    cnt1 = lax.fori_loop(0, _NCHUNK, compact1, jnp.int32(0), unroll=8)
    chosen, c_ab, c_at = bin_select(k_rem_v)
    prefix = prefix * 256 + chosen
    k_rem_v = k_rem_v - c_ab

    zero_hist()
    t1 = lax.shift_right_logical(cnt1 + (_L - 1), 4)

    def compact2(c, cnt):
        k = ck_v[pl.ds(c * _L, _L)]
        inb = (c * _L + lanes) < cnt1
        m = jnp.logical_and(inb, lax.shift_right_logical(k, 16) == prefix)
        mi = m.astype(jnp.int32)
        pos = cnt + plsc.cumsum(mi) - 1
        plsc.store_scatter(ck2_v, [pos], k, mask=m)
        plsc.addupdate_scatter(
            hist_v,
            [lane_base + jnp.bitwise_and(lax.shift_right_logical(k, 8),
                                         255)],
            mi)
        return cnt + jnp.sum(mi)

    cnt2 = lax.fori_loop(0, t1, compact2, jnp.int32(0))
    chosen, c_ab, c_at = bin_select(k_rem_v)
    prefix = prefix * 256 + chosen
    k_rem_v = k_rem_v - c_ab

    zero_hist()
    t2 = lax.shift_right_logical(cnt2 + (_L - 1), 4)

    def scan4(c, _):
        k = ck2_v[pl.ds(c * _L, _L)]
        inb = (c * _L + lanes) < cnt2
        m = jnp.logical_and(inb, lax.shift_right_logical(k, 8) == prefix)
        plsc.addupdate_scatter(hist_v, [lane_base + jnp.bitwise_and(k, 255)],
                               m.astype(jnp.int32))
        return 0

    lax.fori_loop(0, t2, scan4, 0)
    chosen, c_ab, c_at = bin_select(k_rem_v)
    prefix = prefix * 256 + chosen
    k_rem_v = k_rem_v - c_ab

    kt = prefix ^ jnp.int32(_MIN32)
    tb = jnp.where(kt < 0, kt ^ jnp.int32(0x7FFFFFFF), kt)
    t_fv = lax.bitcast_convert_type(tb, jnp.float32)

    inv_tau = 1.0 / tau_v[...]
    smax_v = jnp.broadcast_to(smax, (_L,))

    def pass_e(c, carry):
        se, ser, ee, eer = carry
        s = s_v[pl.ds(c * _L, _L)]
        rv = r_v[pl.ds(c * _L, _L)]
        e = jnp.exp((s - smax_v) * inv_tau)
        er = e * rv
        zf = jnp.zeros((_L,), jnp.float32)
        gt = s > t_fv
        eq = s == t_fv
        return (se + jnp.where(gt, e, zf), ser + jnp.where(gt, er, zf),
                ee + jnp.where(eq, e, zf), eer + jnp.where(eq, er, zf))

    zf = jnp.zeros((_L,), jnp.float32)
    se, ser, ee, eer = lax.fori_loop(0, _NCHUNK, pass_e, (zf, zf, zf, zf),
                                     unroll=8)
    frac = (k_rem_v.astype(jnp.float32) /
            jnp.maximum(c_at, 1).astype(jnp.float32))
    den = bv(jnp.sum(se)) + frac * bv(jnp.sum(ee))
    num = bv(jnp.sum(ser)) + frac * bv(jnp.sum(eer))
    pred = num / den

    base = jnp.clip(pred, 0.0001, 1 - 0.0001)
    zq = base / (1.0 - base)
    bi = lax.bitcast_convert_type(zq, jnp.int32)
    ex = lax.shift_right_logical(bi, 23) - 127
    man = lax.bitcast_convert_type(jnp.bitwise_or(jnp.bitwise_and(
        bi, jnp.int32(0x7FFFFF)), jnp.int32(0x3F800000)), jnp.float32)
    big = man > 1.4142135
    man = jnp.where(big, man * 0.5, man)
    ex = ex + jnp.where(big, 1, 0)
    u = (man - 1.0) / (man + 1.0)
    u2 = u * u
    lnm = 2.0 * u * (1.0 + u2 * (1.0 / 3.0 + u2 * (
        1.0 / 5.0 + u2 * (1.0 / 7.0 + u2 * (1.0 / 9.0)))))
    logit = ex.astype(jnp.float32) * 0.6931471805599453 + lnm
    zz = sc_v[...] * logit + rs_v[...]
    res_v[...] = 1.0 / (1.0 + jnp.exp(-zz))
    pltpu.sync_copy(res_v, out_hbm.at[wid])


def _sc_tail(scores, ref_vals, tau, res_scale, scale):
    mesh = plsc.VectorSubcoreMesh(core_axis_name="c", subcore_axis_name="s")
    f = pl.kernel(
        _sc_body,
        out_type=jax.ShapeDtypeStruct((_B, _L), jnp.float32),
        mesh=mesh,
        compiler_params=pltpu.CompilerParams(needs_layout_passes=False),
        scratch_types=[
            pltpu.VMEM((_N,), jnp.float32),
            pltpu.VMEM((_N,), jnp.float32),
            pltpu.VMEM((_N,), jnp.int32),
            pltpu.VMEM((_N,), jnp.int32),
            pltpu.VMEM((_N,), jnp.int32),
            pltpu.VMEM((256 * _L,), jnp.int32),
            pltpu.VMEM((_L,), jnp.float32),
            pltpu.VMEM((_L,), jnp.float32),
            pltpu.VMEM((_L,), jnp.float32),
            pltpu.VMEM((_L,), jnp.float32),
            pltpu.SemaphoreType.DMA,
        ],
    )
    tv = jnp.broadcast_to(tau.reshape(1), (_L,))
    rsv = jnp.broadcast_to(res_scale.reshape(1), (_L,))
    scv = jnp.broadcast_to(scale.reshape(1), (_L,))
    return f(scores, ref_vals, tv, rsv, scv)


@jax.jit
def kernel(q, r, ref_vals, tau,
           q_ln1_g, q_ln1_b, q_W, q_b, q_ln2_g, q_ln2_b,
           r_ln1_g, r_ln1_b, r_W, r_b, r_ln2_g, r_ln2_b,
           res_scale, scale):
    scores = _tc_scores(q, r, q_ln1_g, q_ln1_b, q_W, q_b, q_ln2_g, q_ln2_b,
                        r_ln1_g, r_ln1_b, r_W, r_b, r_ln2_g,
                        r_ln2_b).reshape(_B, _N)
    out = _sc_tail(scores, ref_vals, tau, res_scale, scale)
    return out[:, 0]
